# baseline (speedup 1.0000x reference)
import functools

import jax
import jax.numpy as jnp
from jax import lax
from jax.experimental import pallas as pl
from jax.experimental.pallas import tpu as pltpu
from jax.experimental.pallas import tpu_sc as plsc

N = 10000
E = 320000
NC = 2
NS = 16
NW = NC * NS
EPW = E // NW
CHUNK = 80
NCHUNK = EPW // CHUNK
NBUF = 5
NROUND = NCHUNK // NBUF
NP = 10240
RPT = NP // NS

_SELU_ALPHA = 1.6732632423543772
_SELU_SCALE = 1.0507009873554805


def _make_prop(D, dtype):
    mesh = plsc.VectorSubcoreMesh(core_axis_name="c", subcore_axis_name="s")
    lanes = 32 if dtype == jnp.bfloat16 else 16

    @functools.partial(
        pl.kernel,
        mesh=mesh,
        compiler_params=pltpu.CompilerParams(use_tc_tiling_on_sc=False),
        out_type=jax.ShapeDtypeStruct((NC, NP, D), dtype),
        scratch_types=[
            pltpu.VMEM((NCHUNK, CHUNK), jnp.int32),
            pltpu.VMEM((NCHUNK, CHUNK), jnp.int32),
            [pltpu.VMEM((CHUNK, D), dtype) for _ in range(NBUF)],
            pltpu.VMEM_SHARED((NP, D), dtype),
            [pltpu.SemaphoreType.DMA for _ in range(NBUF)],
            [pltpu.SemaphoreType.DMA for _ in range(NBUF)],
        ],
    )
    def prop(h_hbm, src_hbm, dst_hbm, out_hbm, srcb, dstb, rows, acc,
             gsem, ssem):
        cid = lax.axis_index("c")
        sid = lax.axis_index("s")
        wid = cid * NS + sid
        pltpu.sync_copy(src_hbm.at[wid], srcb)
        pltpu.sync_copy(dst_hbm.at[wid], dstb)
        zvec = jnp.zeros((lanes,), dtype)

        def zbody(r, carry):
            for c in range(D // lanes):
                rows[0][r, pl.ds(c * lanes, lanes)] = zvec
            return carry

        lax.fori_loop(0, CHUNK, zbody, 0)
        for k in range(RPT // CHUNK):
            pltpu.sync_copy(rows[0], acc.at[pl.ds(sid * RPT + k * CHUNK, CHUNK)])
        plsc.subcore_barrier()

        def gather_start(i, b):
            pltpu.async_copy(h_hbm.at[srcb.at[i]], rows[b], gsem[b])

        def gather_wait(i, b):
            pltpu.make_async_copy(h_hbm.at[srcb.at[i]], rows[b], gsem[b]).wait()

        def scat_start(i, b):
            pltpu.async_copy(rows[b], acc.at[dstb.at[i]], ssem[b], add=True)

        def scat_wait(i, b):
            pltpu.make_async_copy(rows[b], acc.at[dstb.at[i]], ssem[b]).wait()

        for b in range(NBUF):
            gather_start(b, b)

        def round_body(r, carry):
            i = r * NBUF
            for b in range(NBUF):
                gather_wait(i + b, b)
                scat_start(i + b, b)
            for b in range(NBUF):
                scat_wait(i + b, b)
                gather_start(i + NBUF + b, b)
            return carry

        lax.fori_loop(0, NROUND - 1, round_body, 0)
        i = (NROUND - 1) * NBUF
        for b in range(NBUF):
            gather_wait(i + b, b)
            scat_start(i + b, b)
        for b in range(NBUF):
            scat_wait(i + b, b)
        plsc.subcore_barrier()
        pltpu.sync_copy(
            acc.at[pl.ds(sid * RPT, RPT)],
            out_hbm.at[cid, pl.ds(sid * RPT, RPT)],
        )

    return prop


_prop128 = _make_prop(128, jnp.bfloat16)
_prop64 = _make_prop(64, jnp.float32)

_GB = 10
_RB = NP // _GB
_PB = _RB // 2


def _tobf16(xf):
    n = xf.shape[0]
    blk = n // _GB

    def body(a_ref, o_ref):
        o_ref[...] = a_ref[...].astype(jnp.bfloat16)

    return pl.pallas_call(
        body,
        grid=(_GB,),
        in_specs=[pl.BlockSpec((blk,), lambda i: (i,))],
        out_specs=pl.BlockSpec((blk,), lambda i: (i,)),
        out_shape=jax.ShapeDtypeStruct((n,), jnp.bfloat16),
    )(xf)


def _combine(p):
    blk = NP * 128 // _GB

    def body(a_ref, b_ref, o_ref):
        s = a_ref[...].astype(jnp.float32) + b_ref[...].astype(jnp.float32)
        o_ref[...] = s.astype(jnp.bfloat16)

    return pl.pallas_call(
        body,
        grid=(_GB,),
        in_specs=[
            pl.BlockSpec((blk,), lambda i: (i,)),
            pl.BlockSpec((blk,), lambda i: (i + _GB,)),
        ],
        out_specs=pl.BlockSpec((blk,), lambda i: (i,)),
        out_shape=jax.ShapeDtypeStruct((NP * 128,), jnp.bfloat16),
    )(p, p)


def _mlp(p2, W1p, b1p, W2p):
    blk = NP * 128 // _GB

    def body(a_ref, b_ref, w1_ref, b1_ref, w2_ref, o_ref):
        h = (a_ref[...].astype(jnp.float32)
             + b_ref[...].astype(jnp.float32)).reshape(_PB, 256)
        h = jnp.dot(h, w1_ref[...], preferred_element_type=jnp.float32)
        h = h + b1_ref[...]
        h = _SELU_SCALE * jnp.where(h > 0, h, _SELU_ALPHA * (jnp.exp(h) - 1.0))
        o_ref[...] = jnp.dot(h, w2_ref[...], preferred_element_type=jnp.float32)

    return pl.pallas_call(
        body,
        grid=(_GB,),
        in_specs=[
            pl.BlockSpec((blk,), lambda i: (i,)),
            pl.BlockSpec((blk,), lambda i: (i + _GB,)),
            pl.BlockSpec((256, 256), lambda i: (0, 0)),
            pl.BlockSpec((1, 256), lambda i: (0, 0)),
            pl.BlockSpec((256, 128), lambda i: (0, 0)),
        ],
        out_specs=pl.BlockSpec((_PB, 128), lambda i: (i, 0)),
        out_shape=jax.ShapeDtypeStruct((NP // 2, 128), jnp.float32),
    )(p2, p2, W1p, b1p, W2p)


def _final(p3, gp, b2p):
    def body(a_ref, b_ref, g_ref, b2_ref, o_ref):
        s = a_ref[0] + b_ref[0] + g_ref[...] + b2_ref[...]
        lo = s[:, :64]
        hi = s[:, 64:]

        def lsm(t):
            t = t - jnp.max(t, axis=1, keepdims=True)
            return t - jnp.log(jnp.sum(jnp.exp(t), axis=1, keepdims=True))

        out = jnp.stack([lsm(lo), lsm(hi)], axis=1)
        o_ref[...] = out.reshape(_RB, 64)

    return pl.pallas_call(
        body,
        grid=(_GB,),
        in_specs=[
            pl.BlockSpec((1, _PB, 128), lambda i: (0, i, 0)),
            pl.BlockSpec((1, _PB, 128), lambda i: (1, i, 0)),
            pl.BlockSpec((_PB, 128), lambda i: (i, 0)),
            pl.BlockSpec((1, 128), lambda i: (0, 0)),
        ],
        out_specs=pl.BlockSpec((_RB, 64), lambda i: (i, 0)),
        out_shape=jax.ShapeDtypeStruct((NP, 64), jnp.float32),
    )(p3, p3, gp, b2p)


def kernel(x, edge_index, W1, b1, W2, b2):
    src = edge_index[0].reshape(NW, NCHUNK, CHUNK)
    dst = edge_index[1].reshape(NW, NCHUNK, CHUNK)
    z1 = jnp.zeros((128, 128), jnp.float32)
    z2 = jnp.zeros((128, 64), jnp.float32)
    W1p = jnp.concatenate(
        [jnp.concatenate([W1, z1], axis=1), jnp.concatenate([z1, W1], axis=1)],
        axis=0)
    W2p = jnp.concatenate(
        [jnp.concatenate([W2, z2], axis=1), jnp.concatenate([z2, W2], axis=1)],
        axis=0)
    b1p = jnp.concatenate([b1, b1]).reshape(1, 256)
    b2p = jnp.concatenate([b2, b2]).reshape(1, 128)

    xb = _tobf16(x.reshape(N * 128))
    p1 = _prop128(xb.reshape(N, 128), src, dst)
    h1 = _combine(p1.reshape(NC * NP * 128))
    p2 = _prop128(h1.reshape(NP, 128), src, dst)
    gp = _mlp(p2.reshape(NC * NP * 128), W1p, b1p, W2p)
    p3 = _prop64(gp.reshape(NP, 64), src, dst)
    out = _final(p3.reshape(NC, NP // 2, 128), gp, b2p)
    return out[:N]


# speedup vs baseline: 1.0371x; 1.0371x over previous
import functools

import jax
import jax.numpy as jnp
from jax import lax
from jax.experimental import pallas as pl
from jax.experimental.pallas import tpu as pltpu
from jax.experimental.pallas import tpu_sc as plsc

N = 10000
E = 320000
NC = 2
NS = 16
NW = NC * NS
EPW = E // NW
CHUNK = 80
NCHUNK = EPW // CHUNK
NBUF = 5
NROUND = NCHUNK // NBUF
NP = 10240
RPT = NP // NS

_SELU_ALPHA = 1.6732632423543772
_SELU_SCALE = 1.0507009873554805


def _make_prop(D, dtype):
    mesh = plsc.VectorSubcoreMesh(core_axis_name="c", subcore_axis_name="s")
    lanes = 32 if dtype == jnp.bfloat16 else 16

    @functools.partial(
        pl.kernel,
        mesh=mesh,
        compiler_params=pltpu.CompilerParams(use_tc_tiling_on_sc=False),
        out_type=jax.ShapeDtypeStruct((NC, NP, D), dtype),
        scratch_types=[
            pltpu.VMEM((NCHUNK, CHUNK), jnp.int32),
            pltpu.VMEM((NCHUNK, CHUNK), jnp.int32),
            [pltpu.VMEM((CHUNK, D), dtype) for _ in range(NBUF)],
            pltpu.VMEM_SHARED((NP, D), dtype),
            [pltpu.SemaphoreType.DMA for _ in range(NBUF)],
            [pltpu.SemaphoreType.DMA for _ in range(NBUF)],
        ],
    )
    def prop(h_hbm, src_hbm, dst_hbm, out_hbm, srcb, dstb, rows, acc,
             gsem, ssem):
        cid = lax.axis_index("c")
        sid = lax.axis_index("s")
        wid = cid * NS + sid
        pltpu.sync_copy(src_hbm.at[wid], srcb)
        pltpu.sync_copy(dst_hbm.at[wid], dstb)
        zvec = jnp.zeros((lanes,), dtype)

        def zbody(r, carry):
            for c in range(D // lanes):
                rows[0][r, pl.ds(c * lanes, lanes)] = zvec
            return carry

        lax.fori_loop(0, CHUNK, zbody, 0)
        for k in range(RPT // CHUNK):
            pltpu.sync_copy(rows[0], acc.at[pl.ds(sid * RPT + k * CHUNK, CHUNK)])
        plsc.subcore_barrier()

        def gather_start(i, b):
            pltpu.async_copy(h_hbm.at[srcb.at[i]], rows[b], gsem[b])

        def gather_wait(i, b):
            pltpu.make_async_copy(h_hbm.at[srcb.at[i]], rows[b], gsem[b]).wait()

        def scat_start(i, b):
            pltpu.async_copy(rows[b], acc.at[dstb.at[i]], ssem[b], add=True)

        def scat_wait(i, b):
            pltpu.make_async_copy(rows[b], acc.at[dstb.at[i]], ssem[b]).wait()

        for b in range(NBUF):
            gather_start(b, b)

        def round_body(r, carry):
            i = r * NBUF
            for b in range(NBUF):
                gather_wait(i + b, b)
                scat_start(i + b, b)
            for b in range(NBUF):
                scat_wait(i + b, b)
                gather_start(i + NBUF + b, b)
            return carry

        lax.fori_loop(0, NROUND - 1, round_body, 0)
        i = (NROUND - 1) * NBUF
        for b in range(NBUF):
            gather_wait(i + b, b)
            scat_start(i + b, b)
        for b in range(NBUF):
            scat_wait(i + b, b)
        plsc.subcore_barrier()
        pltpu.sync_copy(
            acc.at[pl.ds(sid * RPT, RPT)],
            out_hbm.at[cid, pl.ds(sid * RPT, RPT)],
        )

    return prop


_prop128 = _make_prop(128, jnp.bfloat16)
_prop64 = _make_prop(64, jnp.float32)

_GB = 10
_RB = NP // _GB
_PB = _RB // 2


def _tobf16(xf):
    n = xf.shape[0]
    blk = n // _GB

    def body(a_ref, o_ref):
        o_ref[...] = a_ref[...].astype(jnp.bfloat16)

    return pl.pallas_call(
        body,
        grid=(_GB,),
        in_specs=[pl.BlockSpec((blk,), lambda i: (i,))],
        out_specs=pl.BlockSpec((blk,), lambda i: (i,)),
        out_shape=jax.ShapeDtypeStruct((n,), jnp.bfloat16),
    )(xf)


def _combine(p):
    blk = NP * 128 // _GB

    def body(a_ref, b_ref, o_ref):
        s = a_ref[...].astype(jnp.float32) + b_ref[...].astype(jnp.float32)
        o_ref[...] = s.astype(jnp.bfloat16)

    return pl.pallas_call(
        body,
        grid=(_GB,),
        in_specs=[
            pl.BlockSpec((blk,), lambda i: (i,)),
            pl.BlockSpec((blk,), lambda i: (i + _GB,)),
        ],
        out_specs=pl.BlockSpec((blk,), lambda i: (i,)),
        out_shape=jax.ShapeDtypeStruct((NP * 128,), jnp.bfloat16),
    )(p, p)


def _mlp(p2, W1p, b1p, W2p):
    blk = NP * 128 // _GB

    def body(a_ref, b_ref, w1_ref, b1_ref, w2_ref, o_ref):
        h = (a_ref[...].astype(jnp.float32)
             + b_ref[...].astype(jnp.float32)).reshape(_PB, 256)
        h = jnp.dot(h, w1_ref[...], preferred_element_type=jnp.float32)
        h = h + b1_ref[...]
        h = _SELU_SCALE * jnp.where(h > 0, h, _SELU_ALPHA * (jnp.exp(h) - 1.0))
        o_ref[...] = jnp.dot(h, w2_ref[...], preferred_element_type=jnp.float32)

    return pl.pallas_call(
        body,
        grid=(_GB,),
        in_specs=[
            pl.BlockSpec((blk,), lambda i: (i,)),
            pl.BlockSpec((blk,), lambda i: (i + _GB,)),
            pl.BlockSpec((256, 256), lambda i: (0, 0)),
            pl.BlockSpec((1, 256), lambda i: (0, 0)),
            pl.BlockSpec((256, 128), lambda i: (0, 0)),
        ],
        out_specs=pl.BlockSpec((_PB, 128), lambda i: (i, 0)),
        out_shape=jax.ShapeDtypeStruct((NP // 2, 128), jnp.float32),
    )(p2, p2, W1p, b1p, W2p)


def _final(p3, gp, b2p):
    def body(a_ref, b_ref, g_ref, b2_ref, o_ref):
        s = a_ref[0] + b_ref[0] + g_ref[...] + b2_ref[...]
        lo = s[:, :64]
        hi = s[:, 64:]

        def lsm(t):
            t = t - jnp.max(t, axis=1, keepdims=True)
            return t - jnp.log(jnp.sum(jnp.exp(t), axis=1, keepdims=True))

        out = jnp.stack([lsm(lo), lsm(hi)], axis=1)
        o_ref[...] = out.reshape(_RB, 64)

    return pl.pallas_call(
        body,
        grid=(_GB,),
        in_specs=[
            pl.BlockSpec((1, _PB, 128), lambda i: (0, i, 0)),
            pl.BlockSpec((1, _PB, 128), lambda i: (1, i, 0)),
            pl.BlockSpec((_PB, 128), lambda i: (i, 0)),
            pl.BlockSpec((1, 128), lambda i: (0, 0)),
        ],
        out_specs=pl.BlockSpec((_RB, 64), lambda i: (i, 0)),
        out_shape=jax.ShapeDtypeStruct((NP, 64), jnp.float32),
    )(p3, p3, gp, b2p)


def kernel(x, edge_index, W1, b1, W2, b2):
    src = edge_index[0].reshape(NW, NCHUNK, CHUNK)
    dst = edge_index[1].reshape(NW, NCHUNK, CHUNK)
    z1 = jnp.zeros((128, 128), jnp.float32)
    z2 = jnp.zeros((128, 64), jnp.float32)
    W1p = jnp.concatenate(
        [jnp.concatenate([W1, z1], axis=1), jnp.concatenate([z1, W1], axis=1)],
        axis=0)
    W2p = jnp.concatenate(
        [jnp.concatenate([W2, z2], axis=1), jnp.concatenate([z2, W2], axis=1)],
        axis=0)
    b1p = jnp.concatenate([b1, b1]).reshape(1, 256)
    b2p = jnp.concatenate([b2, b2]).reshape(1, 128)

    xb = x.astype(jnp.bfloat16)
    p1 = _prop128(xb, src, dst)
    h1 = _combine(p1.reshape(NC * NP * 128))
    p2 = _prop128(h1.reshape(NP, 128), src, dst)
    gp = _mlp(p2.reshape(NC * NP * 128), W1p, b1p, W2p)
    p3 = _prop64(gp.reshape(NP, 64), src, dst)
    out = _final(p3.reshape(NC, NP // 2, 128), gp, b2p)
    return out[:N]
